# Initial kernel scaffold; baseline (speedup 1.0000x reference)
import jax
import jax.numpy as jnp
from jax.experimental import pallas as pl


def kernel(metadata_a, metadata_b, x_a, x_b, edge_index_a, edge_index_b, fc1_W, fc1_b, fc2_W, fc2_b, gcn1_W, gcn1_b, gcn2_W, gcn2_b, fcc_W, fcc_b, out_W, out_b):
    raise NotImplementedError("write your pallas kernel here")


# speedup vs baseline: 7.9118x; 7.9118x over previous
import functools

import jax
import jax.numpy as jnp
from jax import lax
from jax.experimental import pallas as pl
from jax.experimental.pallas import tpu as pltpu
from jax.experimental.pallas import tpu_sc as plsc

_N = 10000
_F = 128
_E = 320000
_NC = 2
_NS = 16
_CHUNK = 128
_CPB = 16
_NB = 10
_CPT = _CPB * _NB
_EPAD = _CPT * _CHUNK * _NS
_ACC = 10240
_RPT = _ACC // _NS
_BR = 1000


def _sc_mesh():
    return plsc.VectorSubcoreMesh(core_axis_name="c", subcore_axis_name="s")


@functools.partial(
    pl.kernel,
    mesh=_sc_mesh(),
    out_type=jax.ShapeDtypeStruct((_NC, _ACC, _F), jnp.float32),
    scratch_types=[
        pltpu.VMEM((_CPB, _CHUNK), jnp.int32),
        pltpu.VMEM((_CHUNK, _F), jnp.float32),
        pltpu.VMEM_SHARED((_ACC, _F), jnp.float32),
    ],
)
def _deg_kernel(dst_hbm, z_hbm, ones_hbm, deg_hbm, dst_v, ones_v, acc):
    c = lax.axis_index("c")
    s = lax.axis_index("s")
    wid = c * _NS + s
    pltpu.sync_copy(z_hbm, acc.at[pl.ds(s * _RPT, _RPT)])
    pltpu.sync_copy(ones_hbm, ones_v)
    plsc.subcore_barrier()

    def outer(t, carry):
        pltpu.sync_copy(dst_hbm.at[wid, t], dst_v)

        def body(j, carry2):
            pltpu.sync_copy(ones_v, acc.at[dst_v.at[j]], add=True)
            return carry2

        return lax.fori_loop(0, _CPB, body, carry)

    lax.fori_loop(0, _NB, outer, 0)
    plsc.subcore_barrier()
    pltpu.sync_copy(acc.at[pl.ds(s * _RPT, _RPT)],
                    deg_hbm.at[c, pl.ds(s * _RPT, _RPT)])


@functools.partial(
    pl.kernel,
    mesh=_sc_mesh(),
    out_type=jax.ShapeDtypeStruct((_NC, _ACC, _F), jnp.float32),
    scratch_types=[
        pltpu.VMEM((_CPB, _CHUNK), jnp.int32),
        pltpu.VMEM((_CPB, _CHUNK), jnp.int32),
        pltpu.VMEM((_CHUNK, _F), jnp.float32),
        pltpu.VMEM_SHARED((_ACC, _F), jnp.float32),
        pltpu.SemaphoreType.DMA,
    ],
)
def _agg_kernel(src_hbm, dst_hbm, hs_hbm, z_hbm, out_hbm,
                src_v, dst_v, rows_v, acc, sem):
    c = lax.axis_index("c")
    s = lax.axis_index("s")
    wid = c * _NS + s
    pltpu.sync_copy(z_hbm, acc.at[pl.ds(s * _RPT, _RPT)])
    plsc.subcore_barrier()

    def outer(t, carry):
        pltpu.sync_copy(src_hbm.at[wid, t], src_v)
        pltpu.sync_copy(dst_hbm.at[wid, t], dst_v)

        def body(j, carry2):
            pltpu.async_copy(hs_hbm.at[src_v.at[j]], rows_v, sem).wait()
            pltpu.sync_copy(rows_v, acc.at[dst_v.at[j]], add=True)
            return carry2

        return lax.fori_loop(0, _CPB, body, carry)

    lax.fori_loop(0, _NB, outer, 0)
    plsc.subcore_barrier()
    pltpu.sync_copy(acc.at[pl.ds(s * _RPT, _RPT)],
                    out_hbm.at[c, pl.ds(s * _RPT, _RPT)])


def _pre_body(meta_ref, x_ref, fc1w, fc1b, fc2w, fc2b, g1w, m_ref, p1_ref):
    h = jnp.dot(meta_ref[0], fc1w[...], preferred_element_type=jnp.float32)
    h = jnp.maximum(h + fc1b[...], 0.0)
    h = jnp.dot(h, fc2w[...], preferred_element_type=jnp.float32)
    h = jnp.maximum(h + fc2b[...], 0.0)
    m_ref[0] = h
    p1_ref[0] = jnp.dot(x_ref[0], g1w[...], preferred_element_type=jnp.float32)


def _scale_body(p1_ref, deg_ref, hs_ref):
    dinv = 1.0 / jnp.sqrt(deg_ref[0][:, :1] + 1.0)
    hs_ref[0] = p1_ref[0] * dinv


def _mid_body(p1_ref, agg_ref, deg_ref, g1b, g2w, p2_ref, hs2_ref):
    dinv = 1.0 / jnp.sqrt(deg_ref[0][:, :1] + 1.0)
    h1 = jnp.maximum(dinv * agg_ref[0] + (dinv * dinv) * p1_ref[0] + g1b[...],
                     0.0)
    p2 = jnp.dot(h1, g2w[...], preferred_element_type=jnp.float32)
    p2_ref[0] = p2
    hs2_ref[0] = p2 * dinv


def _post_body(m_ref, p2_ref, agg2_ref, deg_ref, g2b, fccw, fccb, outw, outb,
               o_ref):
    dinv_a = 1.0 / jnp.sqrt(deg_ref[0][:, :1] + 1.0)
    dinv_b = 1.0 / jnp.sqrt(deg_ref[1][:, :1] + 1.0)
    h2a = jnp.maximum(
        dinv_a * agg2_ref[0] + (dinv_a * dinv_a) * p2_ref[0] + g2b[...], 0.0)
    h2b = jnp.maximum(
        dinv_b * agg2_ref[1] + (dinv_b * dinv_b) * p2_ref[1] + g2b[...], 0.0)
    c = jnp.dot(m_ref[0], fccw[0], preferred_element_type=jnp.float32)
    c = c + jnp.dot(m_ref[1], fccw[1], preferred_element_type=jnp.float32)
    c = c + jnp.dot(h2a, fccw[2], preferred_element_type=jnp.float32)
    c = c + jnp.dot(h2b, fccw[3], preferred_element_type=jnp.float32)
    c = jnp.maximum(c + fccb[...], 0.0)
    o = jnp.sum(c * outw[...], axis=1, keepdims=True) + outb[...]
    o_ref[...] = jax.nn.sigmoid(o)


def _full(shape):
    n = len(shape)
    return pl.BlockSpec(shape, lambda *_: (0,) * n)


def kernel(metadata_a, metadata_b, x_a, x_b, edge_index_a, edge_index_b,
           fc1_W, fc1_b, fc2_W, fc2_b, gcn1_W, gcn1_b, gcn2_W, gcn2_b,
           fcc_W, fcc_b, out_W, out_b):
    f32 = jnp.float32
    meta = jnp.stack([metadata_a, metadata_b])
    x = jnp.stack([x_a, x_b])

    pad = _EPAD - _E
    zpad = jnp.zeros((pad,), jnp.int32)
    dpad = jnp.full((pad,), _ACC - 1, jnp.int32)
    sa = jnp.concatenate([edge_index_a[0], zpad])
    da = jnp.concatenate([edge_index_a[1], dpad])
    sb = jnp.concatenate([edge_index_b[0] + _N, zpad])
    db = jnp.concatenate([edge_index_b[1], dpad])
    src_all = jnp.concatenate([sa, sb]).reshape(2 * _NS, _NB, _CPB, _CHUNK)
    dst_all = jnp.concatenate([da, db]).reshape(2 * _NS, _NB, _CPB, _CHUNK)

    ones128 = jnp.ones((_CHUNK, _F), f32)
    z128 = jnp.zeros((_RPT, _F), f32)

    deg = _deg_kernel(dst_all, z128, ones128)

    grid = (2, 10)
    gi_f = pl.BlockSpec((1, _BR, _F), lambda g, i: (g, i, 0))
    gi_m = pl.BlockSpec((1, _BR, 256), lambda g, i: (g, i, 0))
    gi_d = pl.BlockSpec((1, _BR, _F), lambda g, i: (g, i, 0))
    w_ff = pl.BlockSpec((_F, _F), lambda g, i: (0, 0))
    w_mf = pl.BlockSpec((256, _F), lambda g, i: (0, 0))
    w_b = pl.BlockSpec((1, _F), lambda g, i: (0, 0))

    m_out, p1 = pl.pallas_call(
        _pre_body,
        grid=grid,
        in_specs=[gi_m, gi_f, w_mf, w_b, w_ff, w_b, w_ff],
        out_specs=[gi_f, gi_f],
        out_shape=[jax.ShapeDtypeStruct((2, _N, _F), f32)] * 2,
    )(meta, x, fc1_W, fc1_b.reshape(1, -1), fc2_W, fc2_b.reshape(1, -1),
      gcn1_W)

    hs1 = pl.pallas_call(
        _scale_body,
        grid=grid,
        in_specs=[gi_f, gi_d],
        out_specs=gi_f,
        out_shape=jax.ShapeDtypeStruct((2, _N, _F), f32),
    )(p1, deg)

    agg1 = _agg_kernel(src_all, dst_all, hs1.reshape(2 * _N, _F), z128)

    p2, hs2 = pl.pallas_call(
        _mid_body,
        grid=grid,
        in_specs=[gi_f, gi_f, gi_d, w_b, w_ff],
        out_specs=[gi_f, gi_f],
        out_shape=[jax.ShapeDtypeStruct((2, _N, _F), f32)] * 2,
    )(p1, agg1, deg, gcn1_b.reshape(1, -1), gcn2_W)

    agg2 = _agg_kernel(src_all, dst_all, hs2.reshape(2 * _N, _F), z128)

    i_f = pl.BlockSpec((2, _BR, _F), lambda i: (0, i, 0))
    i_d = pl.BlockSpec((2, _BR, _F), lambda i: (0, i, 0))
    out = pl.pallas_call(
        _post_body,
        grid=(10,),
        in_specs=[i_f, i_f, i_f, i_d,
                  _full((1, _F)), _full((4, _F, _F)), _full((1, _F)),
                  _full((1, _F)), _full((1, 1))],
        out_specs=pl.BlockSpec((_BR, 1), lambda i: (i, 0)),
        out_shape=jax.ShapeDtypeStruct((_N, 1), f32),
    )(m_out, p2, agg2, deg, gcn2_b.reshape(1, -1),
      fcc_W.reshape(4, _F, _F), fcc_b.reshape(1, -1), out_W.reshape(1, -1),
      out_b.reshape(1, 1))
    return out


# speedup vs baseline: 8.4345x; 1.0661x over previous
import functools

import jax
import jax.numpy as jnp
from jax import lax
from jax.experimental import pallas as pl
from jax.experimental.pallas import tpu as pltpu
from jax.experimental.pallas import tpu_sc as plsc

_N = 10000
_F = 128
_E = 320000
_NC = 2
_NS = 16
_CHUNK = 128
_CPB = 8
_NB = 20
_CPT = _CPB * _NB
_EPAD = _CPT * _CHUNK * _NS
_ACC = 10240
_RPT = _ACC // _NS
_BR = 1000


def _sc_mesh():
    return plsc.VectorSubcoreMesh(core_axis_name="c", subcore_axis_name="s")


@functools.partial(
    pl.kernel,
    mesh=_sc_mesh(),
    out_type=jax.ShapeDtypeStruct((_NC, _ACC, _F), jnp.float32),
    scratch_types=[
        pltpu.VMEM((_CPB, _CHUNK), jnp.int32),
        pltpu.VMEM((_CHUNK, _F), jnp.float32),
        pltpu.VMEM_SHARED((_ACC, _F), jnp.float32),
        pltpu.SemaphoreType.DMA,
    ],
)
def _deg_kernel(dst_hbm, z_hbm, ones_hbm, deg_hbm, dst_v, ones_v, acc, sem):
    c = lax.axis_index("c")
    s = lax.axis_index("s")
    wid = c * _NS + s
    pltpu.sync_copy(z_hbm, acc.at[pl.ds(s * _RPT, _RPT)])
    pltpu.sync_copy(ones_hbm, ones_v)
    plsc.subcore_barrier()

    def outer(t, carry):
        pltpu.sync_copy(dst_hbm.at[wid, t], dst_v)
        hs = [pltpu.async_copy(ones_v, acc.at[dst_v.at[j]], sem, add=True)
              for j in range(_CPB)]
        for h in hs:
            h.wait()
        return carry

    lax.fori_loop(0, _NB, outer, 0)
    plsc.subcore_barrier()
    pltpu.sync_copy(acc.at[pl.ds(s * _RPT, _RPT)],
                    deg_hbm.at[c, pl.ds(s * _RPT, _RPT)])


@functools.partial(
    pl.kernel,
    mesh=_sc_mesh(),
    out_type=jax.ShapeDtypeStruct((_NC, _ACC, _F), jnp.float32),
    scratch_types=[
        pltpu.VMEM((_CPB, _CHUNK), jnp.int32),
        pltpu.VMEM((_CPB, _CHUNK), jnp.int32),
        pltpu.VMEM((_CHUNK, _F), jnp.float32),
        pltpu.VMEM((_CHUNK, _F), jnp.float32),
        pltpu.VMEM_SHARED((_ACC, _F), jnp.float32),
        pltpu.SemaphoreType.DMA,
        pltpu.SemaphoreType.DMA,
        pltpu.SemaphoreType.DMA,
        pltpu.SemaphoreType.DMA,
    ],
)
def _agg_kernel(src_hbm, dst_hbm, hs_hbm, z_hbm, out_hbm,
                src_v, dst_v, rows0, rows1, acc, sg0, sg1, ss0, ss1):
    c = lax.axis_index("c")
    s = lax.axis_index("s")
    wid = c * _NS + s
    pltpu.sync_copy(z_hbm, acc.at[pl.ds(s * _RPT, _RPT)])
    plsc.subcore_barrier()
    rows = (rows0, rows1)
    sg = (sg0, sg1)
    ss = (ss0, ss1)

    def outer(t, carry):
        pltpu.sync_copy(src_hbm.at[wid, t], src_v)
        pltpu.sync_copy(dst_hbm.at[wid, t], dst_v)
        g = [pltpu.async_copy(hs_hbm.at[src_v.at[j]], rows[j], sg[j])
             for j in range(2)]
        sc = [None, None]
        for j in range(_CPB):
            b = j & 1
            g[b].wait()
            sc[b] = pltpu.async_copy(rows[b], acc.at[dst_v.at[j]], ss[b],
                                     add=True)
            if j + 2 < _CPB:
                sc[b].wait()
                g[b] = pltpu.async_copy(hs_hbm.at[src_v.at[j + 2]], rows[b],
                                        sg[b])
        sc[0].wait()
        sc[1].wait()
        return carry

    lax.fori_loop(0, _NB, outer, 0)
    plsc.subcore_barrier()
    pltpu.sync_copy(acc.at[pl.ds(s * _RPT, _RPT)],
                    out_hbm.at[c, pl.ds(s * _RPT, _RPT)])


def _pre_body(meta_ref, x_ref, fc1w, fc1b, fc2w, fc2b, g1w, m_ref, p1_ref):
    h = jnp.dot(meta_ref[0], fc1w[...], preferred_element_type=jnp.float32)
    h = jnp.maximum(h + fc1b[...], 0.0)
    h = jnp.dot(h, fc2w[...], preferred_element_type=jnp.float32)
    h = jnp.maximum(h + fc2b[...], 0.0)
    m_ref[0] = h
    p1_ref[0] = jnp.dot(x_ref[0], g1w[...], preferred_element_type=jnp.float32)


def _scale_body(p1_ref, deg_ref, hs_ref):
    dinv = 1.0 / jnp.sqrt(deg_ref[0][:, :1] + 1.0)
    hs_ref[0] = p1_ref[0] * dinv


def _mid_body(p1_ref, agg_ref, deg_ref, g1b, g2w, p2_ref, hs2_ref):
    dinv = 1.0 / jnp.sqrt(deg_ref[0][:, :1] + 1.0)
    h1 = jnp.maximum(dinv * agg_ref[0] + (dinv * dinv) * p1_ref[0] + g1b[...],
                     0.0)
    p2 = jnp.dot(h1, g2w[...], preferred_element_type=jnp.float32)
    p2_ref[0] = p2
    hs2_ref[0] = p2 * dinv


def _post_body(m_ref, p2_ref, agg2_ref, deg_ref, g2b, fccw, fccb, outw, outb,
               o_ref):
    dinv_a = 1.0 / jnp.sqrt(deg_ref[0][:, :1] + 1.0)
    dinv_b = 1.0 / jnp.sqrt(deg_ref[1][:, :1] + 1.0)
    h2a = jnp.maximum(
        dinv_a * agg2_ref[0] + (dinv_a * dinv_a) * p2_ref[0] + g2b[...], 0.0)
    h2b = jnp.maximum(
        dinv_b * agg2_ref[1] + (dinv_b * dinv_b) * p2_ref[1] + g2b[...], 0.0)
    c = jnp.dot(m_ref[0], fccw[0], preferred_element_type=jnp.float32)
    c = c + jnp.dot(m_ref[1], fccw[1], preferred_element_type=jnp.float32)
    c = c + jnp.dot(h2a, fccw[2], preferred_element_type=jnp.float32)
    c = c + jnp.dot(h2b, fccw[3], preferred_element_type=jnp.float32)
    c = jnp.maximum(c + fccb[...], 0.0)
    o = jnp.sum(c * outw[...], axis=1, keepdims=True) + outb[...]
    o_ref[...] = jax.nn.sigmoid(o)


def _full(shape):
    n = len(shape)
    return pl.BlockSpec(shape, lambda *_: (0,) * n)


def kernel(metadata_a, metadata_b, x_a, x_b, edge_index_a, edge_index_b,
           fc1_W, fc1_b, fc2_W, fc2_b, gcn1_W, gcn1_b, gcn2_W, gcn2_b,
           fcc_W, fcc_b, out_W, out_b):
    f32 = jnp.float32
    meta = jnp.stack([metadata_a, metadata_b])
    x = jnp.stack([x_a, x_b])

    pad = _EPAD - _E
    zpad = jnp.zeros((pad,), jnp.int32)
    dpad = jnp.full((pad,), _ACC - 1, jnp.int32)
    sa = jnp.concatenate([edge_index_a[0], zpad])
    da = jnp.concatenate([edge_index_a[1], dpad])
    sb = jnp.concatenate([edge_index_b[0] + _N, zpad])
    db = jnp.concatenate([edge_index_b[1], dpad])
    src_all = jnp.concatenate([sa, sb]).reshape(2 * _NS, _NB, _CPB, _CHUNK)
    dst_all = jnp.concatenate([da, db]).reshape(2 * _NS, _NB, _CPB, _CHUNK)

    ones128 = jnp.ones((_CHUNK, _F), f32)
    z128 = jnp.zeros((_RPT, _F), f32)

    deg = _deg_kernel(dst_all, z128, ones128)

    grid = (2, 10)
    gi_f = pl.BlockSpec((1, _BR, _F), lambda g, i: (g, i, 0))
    gi_m = pl.BlockSpec((1, _BR, 256), lambda g, i: (g, i, 0))
    gi_d = pl.BlockSpec((1, _BR, _F), lambda g, i: (g, i, 0))
    w_ff = pl.BlockSpec((_F, _F), lambda g, i: (0, 0))
    w_mf = pl.BlockSpec((256, _F), lambda g, i: (0, 0))
    w_b = pl.BlockSpec((1, _F), lambda g, i: (0, 0))

    m_out, p1 = pl.pallas_call(
        _pre_body,
        grid=grid,
        in_specs=[gi_m, gi_f, w_mf, w_b, w_ff, w_b, w_ff],
        out_specs=[gi_f, gi_f],
        out_shape=[jax.ShapeDtypeStruct((2, _N, _F), f32)] * 2,
    )(meta, x, fc1_W, fc1_b.reshape(1, -1), fc2_W, fc2_b.reshape(1, -1),
      gcn1_W)

    hs1 = pl.pallas_call(
        _scale_body,
        grid=grid,
        in_specs=[gi_f, gi_d],
        out_specs=gi_f,
        out_shape=jax.ShapeDtypeStruct((2, _N, _F), f32),
    )(p1, deg)

    agg1 = _agg_kernel(src_all, dst_all, hs1.reshape(2 * _N, _F), z128)

    p2, hs2 = pl.pallas_call(
        _mid_body,
        grid=grid,
        in_specs=[gi_f, gi_f, gi_d, w_b, w_ff],
        out_specs=[gi_f, gi_f],
        out_shape=[jax.ShapeDtypeStruct((2, _N, _F), f32)] * 2,
    )(p1, agg1, deg, gcn1_b.reshape(1, -1), gcn2_W)

    agg2 = _agg_kernel(src_all, dst_all, hs2.reshape(2 * _N, _F), z128)

    i_f = pl.BlockSpec((2, _BR, _F), lambda i: (0, i, 0))
    i_d = pl.BlockSpec((2, _BR, _F), lambda i: (0, i, 0))
    out = pl.pallas_call(
        _post_body,
        grid=(10,),
        in_specs=[i_f, i_f, i_f, i_d,
                  _full((1, _F)), _full((4, _F, _F)), _full((1, _F)),
                  _full((1, _F)), _full((1, 1))],
        out_specs=pl.BlockSpec((_BR, 1), lambda i: (i, 0)),
        out_shape=jax.ShapeDtypeStruct((_N, 1), f32),
    )(m_out, p2, agg2, deg, gcn2_b.reshape(1, -1),
      fcc_W.reshape(4, _F, _F), fcc_b.reshape(1, -1), out_W.reshape(1, -1),
      out_b.reshape(1, 1))
    return out


# speedup vs baseline: 8.5683x; 1.0159x over previous
import functools

import jax
import jax.numpy as jnp
from jax import lax
from jax.experimental import pallas as pl
from jax.experimental.pallas import tpu as pltpu
from jax.experimental.pallas import tpu_sc as plsc

_N = 10000
_F = 128
_E = 320000
_NC = 2
_NS = 16
_CHUNK = 128
_CPB = 16
_NB = 10
_CPT = _CPB * _NB
_EPAD = _CPT * _CHUNK * _NS
_ACC = 10240
_RPT = _ACC // _NS
_BR = 1000


def _sc_mesh():
    return plsc.VectorSubcoreMesh(core_axis_name="c", subcore_axis_name="s")


@functools.partial(
    pl.kernel,
    mesh=_sc_mesh(),
    out_type=jax.ShapeDtypeStruct((_NC, _ACC, _F), jnp.float32),
    scratch_types=[
        pltpu.VMEM((_CPB, _CHUNK), jnp.int32),
        pltpu.VMEM((_CHUNK, _F), jnp.float32),
        pltpu.VMEM_SHARED((_ACC, _F), jnp.float32),
        pltpu.SemaphoreType.DMA,
    ],
)
def _deg_kernel(dst_hbm, z_hbm, ones_hbm, deg_hbm, dst_v, ones_v, acc, sem):
    c = lax.axis_index("c")
    s = lax.axis_index("s")
    wid = c * _NS + s
    pltpu.sync_copy(z_hbm, acc.at[pl.ds(s * _RPT, _RPT)])
    pltpu.sync_copy(ones_hbm, ones_v)
    plsc.subcore_barrier()

    def outer(t, carry):
        pltpu.sync_copy(dst_hbm.at[wid, t], dst_v)
        hs = [pltpu.async_copy(ones_v, acc.at[dst_v.at[j]], sem, add=True)
              for j in range(_CPB)]
        for h in hs:
            h.wait()
        return carry

    lax.fori_loop(0, _NB, outer, 0)
    plsc.subcore_barrier()
    pltpu.sync_copy(acc.at[pl.ds(s * _RPT, _RPT)],
                    deg_hbm.at[c, pl.ds(s * _RPT, _RPT)])


@functools.partial(
    pl.kernel,
    mesh=_sc_mesh(),
    out_type=jax.ShapeDtypeStruct((_NC, _ACC, _F), jnp.float32),
    scratch_types=[
        pltpu.VMEM((_CPB, _CHUNK), jnp.int32),
        pltpu.VMEM((_CPB, _CHUNK), jnp.int32),
        pltpu.VMEM((_CHUNK, _F), jnp.float32),
        pltpu.VMEM((_CHUNK, _F), jnp.float32),
        pltpu.VMEM_SHARED((_ACC, _F), jnp.float32),
        pltpu.SemaphoreType.DMA,
        pltpu.SemaphoreType.DMA,
        pltpu.SemaphoreType.DMA,
        pltpu.SemaphoreType.DMA,
    ],
)
def _agg_kernel(src_hbm, dst_hbm, hs_hbm, z_hbm, out_hbm,
                src_v, dst_v, rows0, rows1, acc, sg0, sg1, ss0, ss1):
    c = lax.axis_index("c")
    s = lax.axis_index("s")
    wid = c * _NS + s
    pltpu.sync_copy(z_hbm, acc.at[pl.ds(s * _RPT, _RPT)])
    plsc.subcore_barrier()
    rows = (rows0, rows1)
    sg = (sg0, sg1)
    ss = (ss0, ss1)

    def outer(t, carry):
        pltpu.sync_copy(src_hbm.at[wid, t], src_v)
        pltpu.sync_copy(dst_hbm.at[wid, t], dst_v)
        g = [pltpu.async_copy(hs_hbm.at[src_v.at[j]], rows[j], sg[j])
             for j in range(2)]
        sc = [None, None]
        for j in range(_CPB):
            b = j & 1
            g[b].wait()
            sc[b] = pltpu.async_copy(rows[b], acc.at[dst_v.at[j]], ss[b],
                                     add=True)
            if j + 2 < _CPB:
                sc[b].wait()
                g[b] = pltpu.async_copy(hs_hbm.at[src_v.at[j + 2]], rows[b],
                                        sg[b])
        sc[0].wait()
        sc[1].wait()
        return carry

    lax.fori_loop(0, _NB, outer, 0)
    plsc.subcore_barrier()
    pltpu.sync_copy(acc.at[pl.ds(s * _RPT, _RPT)],
                    out_hbm.at[c, pl.ds(s * _RPT, _RPT)])


def _pre_body(meta_ref, x_ref, fc1w, fc1b, fc2w, fc2b, g1w, m_ref, p1_ref):
    h = jnp.dot(meta_ref[0], fc1w[...], preferred_element_type=jnp.float32)
    h = jnp.maximum(h + fc1b[...], 0.0)
    h = jnp.dot(h, fc2w[...], preferred_element_type=jnp.float32)
    h = jnp.maximum(h + fc2b[...], 0.0)
    m_ref[0] = h
    p1_ref[0] = jnp.dot(x_ref[0], g1w[...], preferred_element_type=jnp.float32)


def _scale_body(p1_ref, deg_ref, hs_ref):
    dinv = 1.0 / jnp.sqrt(deg_ref[0][:, :1] + 1.0)
    hs_ref[0] = p1_ref[0] * dinv


def _mid_body(p1_ref, agg_ref, deg_ref, g1b, g2w, p2_ref, hs2_ref):
    dinv = 1.0 / jnp.sqrt(deg_ref[0][:, :1] + 1.0)
    h1 = jnp.maximum(dinv * agg_ref[0] + (dinv * dinv) * p1_ref[0] + g1b[...],
                     0.0)
    p2 = jnp.dot(h1, g2w[...], preferred_element_type=jnp.float32)
    p2_ref[0] = p2
    hs2_ref[0] = p2 * dinv


def _post_body(m_ref, p2_ref, agg2_ref, deg_ref, g2b, fccw, fccb, outw, outb,
               o_ref):
    dinv_a = 1.0 / jnp.sqrt(deg_ref[0][:, :1] + 1.0)
    dinv_b = 1.0 / jnp.sqrt(deg_ref[1][:, :1] + 1.0)
    h2a = jnp.maximum(
        dinv_a * agg2_ref[0] + (dinv_a * dinv_a) * p2_ref[0] + g2b[...], 0.0)
    h2b = jnp.maximum(
        dinv_b * agg2_ref[1] + (dinv_b * dinv_b) * p2_ref[1] + g2b[...], 0.0)
    c = jnp.dot(m_ref[0], fccw[0], preferred_element_type=jnp.float32)
    c = c + jnp.dot(m_ref[1], fccw[1], preferred_element_type=jnp.float32)
    c = c + jnp.dot(h2a, fccw[2], preferred_element_type=jnp.float32)
    c = c + jnp.dot(h2b, fccw[3], preferred_element_type=jnp.float32)
    c = jnp.maximum(c + fccb[...], 0.0)
    o = jnp.sum(c * outw[...], axis=1, keepdims=True) + outb[...]
    o_ref[...] = jax.nn.sigmoid(o)


def _full(shape):
    n = len(shape)
    return pl.BlockSpec(shape, lambda *_: (0,) * n)


def kernel(metadata_a, metadata_b, x_a, x_b, edge_index_a, edge_index_b,
           fc1_W, fc1_b, fc2_W, fc2_b, gcn1_W, gcn1_b, gcn2_W, gcn2_b,
           fcc_W, fcc_b, out_W, out_b):
    f32 = jnp.float32
    meta = jnp.stack([metadata_a, metadata_b])
    x = jnp.stack([x_a, x_b])

    pad = _EPAD - _E
    zpad = jnp.zeros((pad,), jnp.int32)
    dpad = jnp.full((pad,), _ACC - 1, jnp.int32)
    sa = jnp.concatenate([edge_index_a[0], zpad])
    da = jnp.concatenate([edge_index_a[1], dpad])
    sb = jnp.concatenate([edge_index_b[0] + _N, zpad])
    db = jnp.concatenate([edge_index_b[1], dpad])
    src_all = jnp.concatenate([sa, sb]).reshape(2 * _NS, _NB, _CPB, _CHUNK)
    dst_all = jnp.concatenate([da, db]).reshape(2 * _NS, _NB, _CPB, _CHUNK)

    ones128 = jnp.ones((_CHUNK, _F), f32)
    z128 = jnp.zeros((_RPT, _F), f32)

    deg = _deg_kernel(dst_all, z128, ones128)

    grid = (2, 10)
    gi_f = pl.BlockSpec((1, _BR, _F), lambda g, i: (g, i, 0))
    gi_m = pl.BlockSpec((1, _BR, 256), lambda g, i: (g, i, 0))
    gi_d = pl.BlockSpec((1, _BR, _F), lambda g, i: (g, i, 0))
    w_ff = pl.BlockSpec((_F, _F), lambda g, i: (0, 0))
    w_mf = pl.BlockSpec((256, _F), lambda g, i: (0, 0))
    w_b = pl.BlockSpec((1, _F), lambda g, i: (0, 0))

    m_out, p1 = pl.pallas_call(
        _pre_body,
        grid=grid,
        in_specs=[gi_m, gi_f, w_mf, w_b, w_ff, w_b, w_ff],
        out_specs=[gi_f, gi_f],
        out_shape=[jax.ShapeDtypeStruct((2, _N, _F), f32)] * 2,
    )(meta, x, fc1_W, fc1_b.reshape(1, -1), fc2_W, fc2_b.reshape(1, -1),
      gcn1_W)

    hs1 = pl.pallas_call(
        _scale_body,
        grid=grid,
        in_specs=[gi_f, gi_d],
        out_specs=gi_f,
        out_shape=jax.ShapeDtypeStruct((2, _N, _F), f32),
    )(p1, deg)

    agg1 = _agg_kernel(src_all, dst_all, hs1.reshape(2 * _N, _F), z128)

    p2, hs2 = pl.pallas_call(
        _mid_body,
        grid=grid,
        in_specs=[gi_f, gi_f, gi_d, w_b, w_ff],
        out_specs=[gi_f, gi_f],
        out_shape=[jax.ShapeDtypeStruct((2, _N, _F), f32)] * 2,
    )(p1, agg1, deg, gcn1_b.reshape(1, -1), gcn2_W)

    agg2 = _agg_kernel(src_all, dst_all, hs2.reshape(2 * _N, _F), z128)

    i_f = pl.BlockSpec((2, _BR, _F), lambda i: (0, i, 0))
    i_d = pl.BlockSpec((2, _BR, _F), lambda i: (0, i, 0))
    out = pl.pallas_call(
        _post_body,
        grid=(10,),
        in_specs=[i_f, i_f, i_f, i_d,
                  _full((1, _F)), _full((4, _F, _F)), _full((1, _F)),
                  _full((1, _F)), _full((1, 1))],
        out_specs=pl.BlockSpec((_BR, 1), lambda i: (i, 0)),
        out_shape=jax.ShapeDtypeStruct((_N, 1), f32),
    )(m_out, p2, agg2, deg, gcn2_b.reshape(1, -1),
      fcc_W.reshape(4, _F, _F), fcc_b.reshape(1, -1), out_W.reshape(1, -1),
      out_b.reshape(1, 1))
    return out
